# Initial kernel scaffold; baseline (speedup 1.0000x reference)
import jax
import jax.numpy as jnp
from jax.experimental import pallas as pl


def kernel(x_src, x_dst, ln_g_src, ln_b_src, ln_g_dst, ln_b_dst, W_self_src, W_nbr_src, W_self_dst, W_nbr_dst, edge_index):
    raise NotImplementedError("write your pallas kernel here")


# speedup vs baseline: 7.9370x; 7.9370x over previous
import jax
import jax.numpy as jnp
from jax import lax
from jax.experimental import pallas as pl
from jax.experimental.pallas import tpu as pltpu
from jax.experimental.pallas import tpu_sc as plsc

N = 10000
E = 320000
D = 128
NS = 16
EPT = E // NS
B = 80
NB = EPT // B
IC = 10
NCH = NB // IC
NPAD = 10240
RPT = NPAD // NS
CW = 16


def _ln_relu_body(xs_ref, xd_ref, gs_ref, bs_ref, gd_ref, bd_ref,
                  hs_ref, hd_ref):
    for x_ref, g_ref, b_ref, h_ref in (
        (xs_ref, gs_ref, bs_ref, hs_ref),
        (xd_ref, gd_ref, bd_ref, hd_ref),
    ):
        x = x_ref[...]
        m = jnp.mean(x, axis=-1, keepdims=True)
        v = jnp.mean(jnp.square(x - m), axis=-1, keepdims=True)
        h = (x - m) * jax.lax.rsqrt(v + 1e-5) * g_ref[...] + b_ref[...]
        h_ref[...] = jnp.maximum(h, 0.0)


def _sc_agg_body(hs_ref, hd_ref, ei_ref, z_rows_ref, z_cnt_ref, one_cnt_ref,
                 msg_d_ref, cnt_d_ref, msg_s_ref, cnt_s_ref,
                 acc_msg, acc_cnt, gidx, sidx, rows, ones,
                 sem0, sem1):
    s = lax.axis_index("s")

    def run(h_ref, gsel, ssel, msg_out, cnt_out):
        pltpu.sync_copy(one_cnt_ref, ones)
        pltpu.sync_copy(z_rows_ref, acc_msg.at[pl.ds(s * RPT, RPT)])
        pltpu.sync_copy(z_cnt_ref, acc_cnt.at[pl.ds(s * RPT, RPT)])
        plsc.subcore_barrier()

        def chunk(ci, carry):
            pltpu.sync_copy(ei_ref.at[gsel, s, ci], gidx)
            pltpu.sync_copy(ei_ref.at[ssel, s, ci], sidx)
            for k in range(IC // 2):
                j0 = 2 * k
                j1 = 2 * k + 1
                d0 = pltpu.async_copy(h_ref.at[gidx.at[j0]], rows.at[0], sem0)
                d1 = pltpu.async_copy(h_ref.at[gidx.at[j1]], rows.at[1], sem1)
                d0.wait()
                pltpu.sync_copy(rows.at[0], acc_msg.at[sidx.at[j0]], add=True)
                pltpu.sync_copy(ones, acc_cnt.at[sidx.at[j0]], add=True)
                d1.wait()
                pltpu.sync_copy(rows.at[1], acc_msg.at[sidx.at[j1]], add=True)
                pltpu.sync_copy(ones, acc_cnt.at[sidx.at[j1]], add=True)
            return carry

        lax.fori_loop(0, NCH, chunk, 0)
        plsc.subcore_barrier()
        sl = pl.ds(s * RPT, RPT)
        pltpu.sync_copy(acc_msg.at[sl], msg_out.at[sl])
        pltpu.sync_copy(acc_cnt.at[sl], cnt_out.at[sl])

    c = lax.axis_index("c")

    @pl.when(c == 0)
    def _():
        run(hs_ref, 0, 1, msg_d_ref, cnt_d_ref)

    @pl.when(c == 1)
    def _():
        run(hd_ref, 1, 0, msg_s_ref, cnt_s_ref)


def _combine_body(xs_ref, xd_ref, hs_ref, hd_ref,
                  msg_d_ref, cnt_d_ref, msg_s_ref, cnt_s_ref,
                  wss_ref, wns_ref, wsd_ref, wnd_ref,
                  ys_ref, yd_ref):
    inv_d = 1.0 / jnp.maximum(cnt_d_ref[:, :1], 1.0)
    inv_s = 1.0 / jnp.maximum(cnt_s_ref[:, :1], 1.0)
    f32 = jnp.float32
    yd = jnp.dot(hd_ref[...], wsd_ref[...], preferred_element_type=f32)
    yd += jnp.dot(msg_d_ref[...], wnd_ref[...], preferred_element_type=f32) * inv_d
    yd_ref[...] = xd_ref[...] + yd
    ys = jnp.dot(hs_ref[...], wss_ref[...], preferred_element_type=f32)
    ys += jnp.dot(msg_s_ref[...], wns_ref[...], preferred_element_type=f32) * inv_s
    ys_ref[...] = xs_ref[...] + ys


def _make_sc_agg():
    mesh = plsc.VectorSubcoreMesh(core_axis_name="c", subcore_axis_name="s")
    return pl.kernel(
        _sc_agg_body,
        out_type=(
            jax.ShapeDtypeStruct((NPAD, D), jnp.float32),
            jax.ShapeDtypeStruct((NPAD, CW), jnp.float32),
            jax.ShapeDtypeStruct((NPAD, D), jnp.float32),
            jax.ShapeDtypeStruct((NPAD, CW), jnp.float32),
        ),
        mesh=mesh,
        compiler_params=pltpu.CompilerParams(use_tc_tiling_on_sc=False),
        scratch_types=[
            pltpu.VMEM_SHARED((NPAD, D), jnp.float32),
            pltpu.VMEM_SHARED((NPAD, CW), jnp.float32),
            pltpu.VMEM((IC, B), jnp.int32),
            pltpu.VMEM((IC, B), jnp.int32),
            pltpu.VMEM((2, B, D), jnp.float32),
            pltpu.VMEM((B, CW), jnp.float32),
            pltpu.SemaphoreType.DMA,
            pltpu.SemaphoreType.DMA,
        ],
    )


def kernel(x_src, x_dst, ln_g_src, ln_b_src, ln_g_dst, ln_b_dst,
           W_self_src, W_nbr_src, W_self_dst, W_nbr_dst, edge_index):
    f32 = jnp.float32
    RB = 2000
    G = N // RB

    gs = ln_g_src.reshape(1, D)
    bs = ln_b_src.reshape(1, D)
    gd = ln_g_dst.reshape(1, D)
    bd = ln_b_dst.reshape(1, D)

    row_spec = pl.BlockSpec((RB, D), lambda i: (i, 0))
    vec_spec = pl.BlockSpec((1, D), lambda i: (0, 0))
    h_src, h_dst = pl.pallas_call(
        _ln_relu_body,
        grid=(G,),
        in_specs=[row_spec, row_spec, vec_spec, vec_spec, vec_spec, vec_spec],
        out_specs=[row_spec, row_spec],
        out_shape=[jax.ShapeDtypeStruct((N, D), f32)] * 2,
    )(x_src, x_dst, gs, bs, gd, bd)

    ei = edge_index.reshape(2, NS, NCH, IC, B)
    z_rows = jnp.zeros((RPT, D), f32)
    z_cnt = jnp.zeros((RPT, CW), f32)
    one_cnt = jnp.ones((B, CW), f32)
    msg_d, cnt_d, msg_s, cnt_s = _make_sc_agg()(
        h_src, h_dst, ei, z_rows, z_cnt, one_cnt)

    cnt_spec = pl.BlockSpec((RB, CW), lambda i: (i, 0))
    w_spec = pl.BlockSpec((D, D), lambda i: (0, 0))
    y_src, y_dst = pl.pallas_call(
        _combine_body,
        grid=(G,),
        in_specs=[row_spec, row_spec, row_spec, row_spec,
                  row_spec, cnt_spec, row_spec, cnt_spec,
                  w_spec, w_spec, w_spec, w_spec],
        out_specs=[row_spec, row_spec],
        out_shape=[jax.ShapeDtypeStruct((N, D), f32)] * 2,
    )(x_src, x_dst, h_src, h_dst,
      msg_d, cnt_d, msg_s, cnt_s,
      W_self_src, W_nbr_src, W_self_dst, W_nbr_dst)

    return (y_src, y_dst)
